# baseline (speedup 1.0000x reference)
import functools

import jax
import jax.numpy as jnp
from jax import lax
from jax.experimental import pallas as pl
from jax.experimental.pallas import tpu as pltpu
from jax.experimental.pallas import tpu_sc as plsc

_NC = 2
_NS = 16
_NW = _NC * _NS


def _sc_gather(table, idx, n_rows):
    per_w = n_rows // _NW
    ch = 40
    n_ch = per_w // ch
    assert per_w % ch == 0 and ch % 8 == 0

    mesh = plsc.VectorSubcoreMesh(core_axis_name="c", subcore_axis_name="s")

    @functools.partial(
        pl.kernel,
        mesh=mesh,
        out_type=jax.ShapeDtypeStruct((n_rows, 8, 128), jnp.float32),
        scratch_types=[
            pltpu.VMEM((ch,), jnp.int32),
            pltpu.VMEM((ch,), jnp.int32),
            pltpu.VMEM((ch, 8, 128), jnp.float32),
            pltpu.VMEM((ch, 8, 128), jnp.float32),
            pltpu.SemaphoreType.DMA,
            pltpu.SemaphoreType.DMA,
            pltpu.SemaphoreType.DMA,
            pltpu.SemaphoreType.DMA,
        ],
    )
    def gather_kernel(table_hbm, idx_hbm, out_hbm, idx0, idx1, buf0, buf1,
                      gsem0, gsem1, wsem0, wsem1):
        wid = lax.axis_index("s") * _NC + lax.axis_index("c")
        base = wid * per_w
        idx_b = [idx0, idx1]
        buf_b = [buf0, buf1]
        gsem_b = [gsem0, gsem1]
        wsem_b = [wsem0, wsem1]

        pltpu.sync_copy(idx_hbm.at[pl.ds(base, ch)], idx0)
        pltpu.async_copy(table_hbm.at[idx0], buf0, gsem0)

        def body(i, carry):
            for p in range(2):
                cur = p
                nxt = 1 - p
                c_id = i * 2 + p

                @pl.when(c_id + 1 < n_ch)
                def _():
                    off = base + (c_id + 1) * ch
                    pltpu.sync_copy(idx_hbm.at[pl.ds(off, ch)], idx_b[nxt])
                    @pl.when(c_id >= 1)
                    def _():
                        pltpu.make_async_copy(
                            buf_b[nxt], out_hbm.at[pl.ds(0, ch)],
                            wsem_b[nxt]).wait()
                    pltpu.async_copy(table_hbm.at[idx_b[nxt]], buf_b[nxt],
                                     gsem_b[nxt])

                pltpu.make_async_copy(
                    table_hbm.at[idx_b[cur]], buf_b[cur], gsem_b[cur]).wait()
                off = base + c_id * ch
                pltpu.async_copy(buf_b[cur], out_hbm.at[pl.ds(off, ch)],
                                 wsem_b[cur])
            return carry

        lax.fori_loop(0, n_ch // 2, body, 0)
        pltpu.make_async_copy(buf0, out_hbm.at[pl.ds(0, ch)], wsem0).wait()
        pltpu.make_async_copy(buf1, out_hbm.at[pl.ds(0, ch)], wsem1).wait()

    return gather_kernel(table, idx)


def _tc_scan(g, s_in, fin, batch, length, s):
    d = s * s
    o = fin.shape[1]

    def scan_kernel(sin_ref, fin_ref, g_ref, r_ref, sfin_ref, s_ref):
        l = pl.program_id(0)

        @pl.when(l == 0)
        def _():
            s_ref[...] = sin_ref[...]

        st = s_ref[...]
        gl = jnp.reshape(g_ref[...], (batch, d))

        row = lax.broadcasted_iota(jnp.int32, (2 * s, d), 0) % s
        colk = lax.broadcasted_iota(jnp.int32, (2 * s, d), 1) // s
        e2 = jnp.where(row == colk, 1.0, 0.0).astype(jnp.float32)
        st_hi = st.astype(jnp.bfloat16).astype(jnp.float32)
        st_lo = st - st_hi
        st2 = jnp.concatenate([st_hi, st_lo], axis=1)
        se = jax.lax.dot_general(
            st2, e2, (((1,), (0,)), ((), ())),
            preferred_element_type=jnp.float32)

        w = gl * se

        w128 = w[:, 0:128]
        for c in range(1, d // 128):
            w128 = w128 + w[:, c * 128:(c + 1) * 128]
        s_new = w128[:, 0:s]
        for q in range(1, 128 // s):
            s_new = s_new + w128[:, q * s:(q + 1) * s]

        s_ref[...] = s_new
        r_ref[0] = jax.lax.dot_general(
            s_new, fin_ref[...], (((1,), (0,)), ((), ())),
            precision=lax.Precision.HIGHEST,
            preferred_element_type=jnp.float32)

        @pl.when(l == length - 1)
        def _():
            sfin_ref[...] = s_new

    return pl.pallas_call(
        scan_kernel,
        grid=(length,),
        in_specs=[
            pl.BlockSpec((batch, s), lambda l: (0, 0)),
            pl.BlockSpec((s, o), lambda l: (0, 0)),
            pl.BlockSpec((batch // 8, 8, 8, 128), lambda l: (l, 0, 0, 0)),
        ],
        out_specs=[
            pl.BlockSpec((1, batch, o), lambda l: (l, 0, 0)),
            pl.BlockSpec((batch, s), lambda l: (0, 0)),
        ],
        out_shape=[
            jax.ShapeDtypeStruct((length, batch, o), jnp.float32),
            jax.ShapeDtypeStruct((batch, s), jnp.float32),
        ],
        scratch_shapes=[pltpu.VMEM((batch, s), jnp.float32)],
        compiler_params=pltpu.CompilerParams(
            dimension_semantics=("arbitrary",)),
    )(s_in, fin, g)


def kernel(action_seq, trans_prob, fin_matrix):
    batch, length = action_seq.shape
    a, s, _ = trans_prob.shape
    d = s * s
    n = length * batch

    table3 = jnp.reshape(trans_prob, (a, 8, d // 8))
    idx = jnp.reshape(jnp.transpose(action_seq, (1, 0)),
                      (n,)).astype(jnp.int32)

    lcs = [15, 15, 15, 5] if length == 50 else [length]

    starts = [sum(lcs[:c]) for c in range(len(lcs))]
    gs = [_sc_gather(table3,
                     lax.slice(idx, (st0 * batch,), ((st0 + lc) * batch,)),
                     lc * batch)
          for st0, lc in zip(starts, lcs)]

    st = jnp.zeros((batch, s), jnp.float32).at[:, 0].set(1.0)
    parts = []
    for g_c, lc in zip(gs, lcs):
        g4 = jnp.reshape(g_c, (lc * batch // 8, 8, 8, d // 8))
        rewards_c, st = _tc_scan(g4, st, fin_matrix, batch, lc, s)
        parts.append(rewards_c)

    rewards = jnp.transpose(jnp.concatenate(parts, axis=0), (1, 0, 2))
    return rewards, st


# speedup vs baseline: 1.0018x; 1.0018x over previous
import functools

import jax
import jax.numpy as jnp
from jax import lax
from jax.experimental import pallas as pl
from jax.experimental.pallas import tpu as pltpu
from jax.experimental.pallas import tpu_sc as plsc

_NC = 2
_NS = 16
_NW = _NC * _NS


def _sc_gather(table, idx, n_rows):
    per_w = n_rows // _NW
    ch = 40
    n_ch = per_w // ch
    assert per_w % ch == 0 and ch % 8 == 0

    mesh = plsc.VectorSubcoreMesh(core_axis_name="c", subcore_axis_name="s")

    @functools.partial(
        pl.kernel,
        mesh=mesh,
        out_type=jax.ShapeDtypeStruct((n_rows, 8, 128), jnp.float32),
        scratch_types=[
            pltpu.VMEM((ch,), jnp.int32),
            pltpu.VMEM((ch,), jnp.int32),
            pltpu.VMEM((ch, 8, 128), jnp.float32),
            pltpu.VMEM((ch, 8, 128), jnp.float32),
            pltpu.SemaphoreType.DMA,
            pltpu.SemaphoreType.DMA,
            pltpu.SemaphoreType.DMA,
            pltpu.SemaphoreType.DMA,
        ],
    )
    def gather_kernel(table_hbm, idx_hbm, out_hbm, idx0, idx1, buf0, buf1,
                      gsem0, gsem1, wsem0, wsem1):
        wid = lax.axis_index("s") * _NC + lax.axis_index("c")
        base = wid * per_w
        idx_b = [idx0, idx1]
        buf_b = [buf0, buf1]
        gsem_b = [gsem0, gsem1]
        wsem_b = [wsem0, wsem1]

        pltpu.sync_copy(idx_hbm.at[pl.ds(base, ch)], idx0)
        pltpu.async_copy(table_hbm.at[idx0], buf0, gsem0)

        def body(i, carry):
            for p in range(2):
                cur = p
                nxt = 1 - p
                c_id = i * 2 + p

                @pl.when(c_id + 1 < n_ch)
                def _():
                    off = base + (c_id + 1) * ch
                    pltpu.sync_copy(idx_hbm.at[pl.ds(off, ch)], idx_b[nxt])
                    @pl.when(c_id >= 1)
                    def _():
                        pltpu.make_async_copy(
                            buf_b[nxt], out_hbm.at[pl.ds(0, ch)],
                            wsem_b[nxt]).wait()
                    pltpu.async_copy(table_hbm.at[idx_b[nxt]], buf_b[nxt],
                                     gsem_b[nxt])

                pltpu.make_async_copy(
                    table_hbm.at[idx_b[cur]], buf_b[cur], gsem_b[cur]).wait()
                off = base + c_id * ch
                pltpu.async_copy(buf_b[cur], out_hbm.at[pl.ds(off, ch)],
                                 wsem_b[cur])
            return carry

        lax.fori_loop(0, n_ch // 2, body, 0)
        pltpu.make_async_copy(buf0, out_hbm.at[pl.ds(0, ch)], wsem0).wait()
        pltpu.make_async_copy(buf1, out_hbm.at[pl.ds(0, ch)], wsem1).wait()

    return gather_kernel(table, idx)


def _tc_scan(g, s_in, fin, batch, length, s):
    d = s * s
    o = fin.shape[1]

    def scan_kernel(sin_ref, fin_ref, g_ref, r_ref, sfin_ref, s_ref):
        l = pl.program_id(0)

        @pl.when(l == 0)
        def _():
            s_ref[...] = sin_ref[...]

        st = s_ref[...]
        gl = jnp.reshape(g_ref[...], (batch, d))

        row = lax.broadcasted_iota(jnp.int32, (2 * s, d), 0) % s
        colk = lax.broadcasted_iota(jnp.int32, (2 * s, d), 1) // s
        e2 = jnp.where(row == colk, 1.0, 0.0).astype(jnp.float32)
        st_hi = st.astype(jnp.bfloat16).astype(jnp.float32)
        st_lo = st - st_hi
        st2 = jnp.concatenate([st_hi, st_lo], axis=1)
        se = jax.lax.dot_general(
            st2, e2, (((1,), (0,)), ((), ())),
            preferred_element_type=jnp.float32)

        w = gl * se

        w128 = w[:, 0:128]
        for c in range(1, d // 128):
            w128 = w128 + w[:, c * 128:(c + 1) * 128]
        s_new = w128[:, 0:s]
        for q in range(1, 128 // s):
            s_new = s_new + w128[:, q * s:(q + 1) * s]

        s_ref[...] = s_new
        r_ref[0] = jax.lax.dot_general(
            s_new, fin_ref[...], (((1,), (0,)), ((), ())),
            precision=lax.Precision.HIGHEST,
            preferred_element_type=jnp.float32)

        @pl.when(l == length - 1)
        def _():
            sfin_ref[...] = s_new

    return pl.pallas_call(
        scan_kernel,
        grid=(length,),
        in_specs=[
            pl.BlockSpec((batch, s), lambda l: (0, 0)),
            pl.BlockSpec((s, o), lambda l: (0, 0)),
            pl.BlockSpec((batch // 8, 8, 8, 128), lambda l: (l, 0, 0, 0)),
        ],
        out_specs=[
            pl.BlockSpec((1, batch, o), lambda l: (l, 0, 0)),
            pl.BlockSpec((batch, s), lambda l: (0, 0)),
        ],
        out_shape=[
            jax.ShapeDtypeStruct((length, batch, o), jnp.float32),
            jax.ShapeDtypeStruct((batch, s), jnp.float32),
        ],
        scratch_shapes=[pltpu.VMEM((batch, s), jnp.float32)],
        compiler_params=pltpu.CompilerParams(
            dimension_semantics=("arbitrary",)),
    )(s_in, fin, g)


def kernel(action_seq, trans_prob, fin_matrix):
    batch, length = action_seq.shape
    a, s, _ = trans_prob.shape
    d = s * s
    n = length * batch

    table3 = jnp.reshape(trans_prob, (a, 8, d // 8))
    idx = jnp.reshape(jnp.transpose(action_seq, (1, 0)),
                      (n,)).astype(jnp.int32)

    lcs = [10, 10, 10, 15, 5] if length == 50 else [length]

    starts = [sum(lcs[:c]) for c in range(len(lcs))]
    gs = [_sc_gather(table3,
                     lax.slice(idx, (st0 * batch,), ((st0 + lc) * batch,)),
                     lc * batch)
          for st0, lc in zip(starts, lcs)]

    st = jnp.zeros((batch, s), jnp.float32).at[:, 0].set(1.0)
    parts = []
    for g_c, lc in zip(gs, lcs):
        g4 = jnp.reshape(g_c, (lc * batch // 8, 8, 8, d // 8))
        rewards_c, st = _tc_scan(g4, st, fin_matrix, batch, lc, s)
        parts.append(rewards_c)

    rewards = jnp.transpose(jnp.concatenate(parts, axis=0), (1, 0, 2))
    return rewards, st


# speedup vs baseline: 1.0060x; 1.0042x over previous
import functools

import jax
import jax.numpy as jnp
from jax import lax
from jax.experimental import pallas as pl
from jax.experimental.pallas import tpu as pltpu
from jax.experimental.pallas import tpu_sc as plsc

_NC = 2
_NS = 16
_NW = _NC * _NS


def _sc_gather(table, idx, n_rows):
    per_w = n_rows // _NW
    ch = 40
    n_ch = per_w // ch
    assert per_w % ch == 0 and ch % 8 == 0

    mesh = plsc.VectorSubcoreMesh(core_axis_name="c", subcore_axis_name="s")

    @functools.partial(
        pl.kernel,
        mesh=mesh,
        out_type=jax.ShapeDtypeStruct((n_rows, 8, 128), jnp.float32),
        scratch_types=[
            pltpu.VMEM((ch,), jnp.int32),
            pltpu.VMEM((ch,), jnp.int32),
            pltpu.VMEM((ch, 8, 128), jnp.float32),
            pltpu.VMEM((ch, 8, 128), jnp.float32),
            pltpu.SemaphoreType.DMA,
            pltpu.SemaphoreType.DMA,
            pltpu.SemaphoreType.DMA,
            pltpu.SemaphoreType.DMA,
        ],
    )
    def gather_kernel(table_hbm, idx_hbm, out_hbm, idx0, idx1, buf0, buf1,
                      gsem0, gsem1, wsem0, wsem1):
        wid = lax.axis_index("s") * _NC + lax.axis_index("c")
        base = wid * per_w
        idx_b = [idx0, idx1]
        buf_b = [buf0, buf1]
        gsem_b = [gsem0, gsem1]
        wsem_b = [wsem0, wsem1]

        pltpu.sync_copy(idx_hbm.at[pl.ds(base, ch)], idx0)
        pltpu.async_copy(table_hbm.at[idx0], buf0, gsem0)

        def body(i, carry):
            for p in range(2):
                cur = p
                nxt = 1 - p
                c_id = i * 2 + p

                @pl.when(c_id + 1 < n_ch)
                def _():
                    off = base + (c_id + 1) * ch
                    pltpu.sync_copy(idx_hbm.at[pl.ds(off, ch)], idx_b[nxt])
                    @pl.when(c_id >= 1)
                    def _():
                        pltpu.make_async_copy(
                            buf_b[nxt], out_hbm.at[pl.ds(0, ch)],
                            wsem_b[nxt]).wait()
                    pltpu.async_copy(table_hbm.at[idx_b[nxt]], buf_b[nxt],
                                     gsem_b[nxt])

                pltpu.make_async_copy(
                    table_hbm.at[idx_b[cur]], buf_b[cur], gsem_b[cur]).wait()
                off = base + c_id * ch
                pltpu.async_copy(buf_b[cur], out_hbm.at[pl.ds(off, ch)],
                                 wsem_b[cur])
            return carry

        lax.fori_loop(0, n_ch // 2, body, 0)
        pltpu.make_async_copy(buf0, out_hbm.at[pl.ds(0, ch)], wsem0).wait()
        pltpu.make_async_copy(buf1, out_hbm.at[pl.ds(0, ch)], wsem1).wait()

    return gather_kernel(table, idx)


def _tc_scan(g, s_in, fin, batch, length, s):
    d = s * s
    o = fin.shape[1]

    def scan_kernel(sin_ref, fin_ref, g_ref, r_ref, sfin_ref, s_ref):
        l = pl.program_id(0)

        @pl.when(l == 0)
        def _():
            s_ref[...] = sin_ref[...]

        st = s_ref[...]
        gl = jnp.reshape(g_ref[...], (batch, d))

        row = lax.broadcasted_iota(jnp.int32, (2 * s, d), 0) % s
        colk = lax.broadcasted_iota(jnp.int32, (2 * s, d), 1) // s
        e2 = jnp.where(row == colk, 1.0, 0.0).astype(jnp.float32)
        st_hi = st.astype(jnp.bfloat16).astype(jnp.float32)
        st_lo = st - st_hi
        st2 = jnp.concatenate([st_hi, st_lo], axis=1)
        se = jax.lax.dot_general(
            st2, e2, (((1,), (0,)), ((), ())),
            preferred_element_type=jnp.float32)

        w = gl * se

        w128 = w[:, 0:128]
        for c in range(1, d // 128):
            w128 = w128 + w[:, c * 128:(c + 1) * 128]
        s_new = w128[:, 0:s]
        for q in range(1, 128 // s):
            s_new = s_new + w128[:, q * s:(q + 1) * s]

        s_ref[...] = s_new
        r_ref[0] = jax.lax.dot_general(
            s_new, fin_ref[...], (((1,), (0,)), ((), ())),
            precision=lax.Precision.HIGHEST,
            preferred_element_type=jnp.float32)

        @pl.when(l == length - 1)
        def _():
            sfin_ref[...] = s_new

    return pl.pallas_call(
        scan_kernel,
        grid=(length,),
        in_specs=[
            pl.BlockSpec((batch, s), lambda l: (0, 0)),
            pl.BlockSpec((s, o), lambda l: (0, 0)),
            pl.BlockSpec((batch // 8, 8, 8, 128), lambda l: (l, 0, 0, 0)),
        ],
        out_specs=[
            pl.BlockSpec((1, batch, o), lambda l: (l, 0, 0)),
            pl.BlockSpec((batch, s), lambda l: (0, 0)),
        ],
        out_shape=[
            jax.ShapeDtypeStruct((length, batch, o), jnp.float32),
            jax.ShapeDtypeStruct((batch, s), jnp.float32),
        ],
        scratch_shapes=[pltpu.VMEM((batch, s), jnp.float32)],
        compiler_params=pltpu.CompilerParams(
            dimension_semantics=("arbitrary",)),
    )(s_in, fin, g)


def kernel(action_seq, trans_prob, fin_matrix):
    batch, length = action_seq.shape
    a, s, _ = trans_prob.shape
    d = s * s
    n = length * batch

    table3 = jnp.reshape(trans_prob, (a, 8, d // 8))
    idx = jnp.reshape(jnp.transpose(action_seq, (1, 0)),
                      (n,)).astype(jnp.int32)

    lcs = [10, 10, 10, 10, 5, 5] if length == 50 else [length]

    starts = [sum(lcs[:c]) for c in range(len(lcs))]
    gs = [_sc_gather(table3,
                     lax.slice(idx, (st0 * batch,), ((st0 + lc) * batch,)),
                     lc * batch)
          for st0, lc in zip(starts, lcs)]

    st = jnp.zeros((batch, s), jnp.float32).at[:, 0].set(1.0)
    parts = []
    for g_c, lc in zip(gs, lcs):
        g4 = jnp.reshape(g_c, (lc * batch // 8, 8, 8, d // 8))
        rewards_c, st = _tc_scan(g4, st, fin_matrix, batch, lc, s)
        parts.append(rewards_c)

    rewards = jnp.transpose(jnp.concatenate(parts, axis=0), (1, 0, 2))
    return rewards, st


# speedup vs baseline: 1.0126x; 1.0065x over previous
import functools

import jax
import jax.numpy as jnp
from jax import lax
from jax.experimental import pallas as pl
from jax.experimental.pallas import tpu as pltpu
from jax.experimental.pallas import tpu_sc as plsc

_NC = 2
_NS = 16
_NW = _NC * _NS


def _sc_gather(table, idx, n_rows):
    per_w = n_rows // _NW
    ch = 40
    n_ch = per_w // ch
    assert per_w % ch == 0 and ch % 8 == 0

    mesh = plsc.VectorSubcoreMesh(core_axis_name="c", subcore_axis_name="s")

    @functools.partial(
        pl.kernel,
        mesh=mesh,
        out_type=jax.ShapeDtypeStruct((n_rows, 8, 128), jnp.float32),
        scratch_types=[
            pltpu.VMEM((ch,), jnp.int32),
            pltpu.VMEM((ch,), jnp.int32),
            pltpu.VMEM((ch, 8, 128), jnp.float32),
            pltpu.VMEM((ch, 8, 128), jnp.float32),
            pltpu.SemaphoreType.DMA,
            pltpu.SemaphoreType.DMA,
            pltpu.SemaphoreType.DMA,
            pltpu.SemaphoreType.DMA,
        ],
    )
    def gather_kernel(table_hbm, idx_hbm, out_hbm, idx0, idx1, buf0, buf1,
                      gsem0, gsem1, wsem0, wsem1):
        wid = lax.axis_index("s") * _NC + lax.axis_index("c")
        base = wid * per_w
        idx_b = [idx0, idx1]
        buf_b = [buf0, buf1]
        gsem_b = [gsem0, gsem1]
        wsem_b = [wsem0, wsem1]

        pltpu.sync_copy(idx_hbm.at[pl.ds(base, ch)], idx0)
        pltpu.async_copy(table_hbm.at[idx0], buf0, gsem0)

        def body(i, carry):
            for p in range(2):
                cur = p
                nxt = 1 - p
                c_id = i * 2 + p

                @pl.when(c_id + 1 < n_ch)
                def _():
                    off = base + (c_id + 1) * ch
                    pltpu.sync_copy(idx_hbm.at[pl.ds(off, ch)], idx_b[nxt])
                    @pl.when(c_id >= 1)
                    def _():
                        pltpu.make_async_copy(
                            buf_b[nxt], out_hbm.at[pl.ds(0, ch)],
                            wsem_b[nxt]).wait()
                    pltpu.async_copy(table_hbm.at[idx_b[nxt]], buf_b[nxt],
                                     gsem_b[nxt])

                pltpu.make_async_copy(
                    table_hbm.at[idx_b[cur]], buf_b[cur], gsem_b[cur]).wait()
                off = base + c_id * ch
                pltpu.async_copy(buf_b[cur], out_hbm.at[pl.ds(off, ch)],
                                 wsem_b[cur])
            return carry

        lax.fori_loop(0, n_ch // 2, body, 0)
        pltpu.make_async_copy(buf0, out_hbm.at[pl.ds(0, ch)], wsem0).wait()
        pltpu.make_async_copy(buf1, out_hbm.at[pl.ds(0, ch)], wsem1).wait()

    return gather_kernel(table, idx)


def _tc_scan(g, s_in, fin, batch, length, s):
    d = s * s
    o = fin.shape[1]

    def scan_kernel(sin_ref, fin_ref, g_ref, r_ref, sfin_ref, s_ref):
        l = pl.program_id(0)

        @pl.when(l == 0)
        def _():
            s_ref[...] = sin_ref[...]

        st = s_ref[...]
        gl = jnp.reshape(g_ref[...], (batch, d))

        row = lax.broadcasted_iota(jnp.int32, (2 * s, d), 0) % s
        colk = lax.broadcasted_iota(jnp.int32, (2 * s, d), 1) // s
        e2 = jnp.where(row == colk, 1.0, 0.0).astype(jnp.float32)
        st_hi = st.astype(jnp.bfloat16).astype(jnp.float32)
        st_lo = st - st_hi
        st2 = jnp.concatenate([st_hi, st_lo], axis=1)
        se = jax.lax.dot_general(
            st2, e2, (((1,), (0,)), ((), ())),
            preferred_element_type=jnp.float32)

        w = gl * se

        w128 = w[:, 0:128]
        for c in range(1, d // 128):
            w128 = w128 + w[:, c * 128:(c + 1) * 128]
        s_new = w128[:, 0:s]
        for q in range(1, 128 // s):
            s_new = s_new + w128[:, q * s:(q + 1) * s]

        s_ref[...] = s_new
        r_ref[0] = jax.lax.dot_general(
            s_new, fin_ref[...], (((1,), (0,)), ((), ())),
            precision=lax.Precision.HIGHEST,
            preferred_element_type=jnp.float32)

        @pl.when(l == length - 1)
        def _():
            sfin_ref[...] = s_new

    return pl.pallas_call(
        scan_kernel,
        grid=(length,),
        in_specs=[
            pl.BlockSpec((batch, s), lambda l: (0, 0)),
            pl.BlockSpec((s, o), lambda l: (0, 0)),
            pl.BlockSpec((batch // 8, 8, 8, 128), lambda l: (l, 0, 0, 0)),
        ],
        out_specs=[
            pl.BlockSpec((1, batch, o), lambda l: (l, 0, 0)),
            pl.BlockSpec((batch, s), lambda l: (0, 0)),
        ],
        out_shape=[
            jax.ShapeDtypeStruct((length, batch, o), jnp.float32),
            jax.ShapeDtypeStruct((batch, s), jnp.float32),
        ],
        scratch_shapes=[pltpu.VMEM((batch, s), jnp.float32)],
        compiler_params=pltpu.CompilerParams(
            dimension_semantics=("arbitrary",)),
    )(s_in, fin, g)


def kernel(action_seq, trans_prob, fin_matrix):
    batch, length = action_seq.shape
    a, s, _ = trans_prob.shape
    d = s * s
    n = length * batch

    table3 = jnp.reshape(trans_prob, (a, 8, d // 8))
    idx = jnp.reshape(jnp.transpose(action_seq, (1, 0)),
                      (n,)).astype(jnp.int32)

    lcs = [10] * 5 if length == 50 else [length]

    starts = [sum(lcs[:c]) for c in range(len(lcs))]
    gs = [_sc_gather(table3,
                     lax.slice(idx, (st0 * batch,), ((st0 + lc) * batch,)),
                     lc * batch)
          for st0, lc in zip(starts, lcs)]

    st = jnp.zeros((batch, s), jnp.float32).at[:, 0].set(1.0)
    parts = []
    for g_c, lc in zip(gs, lcs):
        g4 = jnp.reshape(g_c, (lc * batch // 8, 8, 8, d // 8))
        rewards_c, st = _tc_scan(g4, st, fin_matrix, batch, lc, s)
        parts.append(rewards_c)

    rewards = jnp.transpose(jnp.concatenate(parts, axis=0), (1, 0, 2))
    return rewards, st


# speedup vs baseline: 1.0182x; 1.0056x over previous
import functools

import jax
import jax.numpy as jnp
from jax import lax
from jax.experimental import pallas as pl
from jax.experimental.pallas import tpu as pltpu
from jax.experimental.pallas import tpu_sc as plsc

_NC = 2
_NS = 16
_NW = _NC * _NS


def _sc_gather(table, idx, n_rows):
    per_w = n_rows // _NW
    ch = 40
    n_ch = per_w // ch
    assert per_w % ch == 0 and ch % 8 == 0

    mesh = plsc.VectorSubcoreMesh(core_axis_name="c", subcore_axis_name="s")

    @functools.partial(
        pl.kernel,
        mesh=mesh,
        out_type=jax.ShapeDtypeStruct((n_rows, 8, 128), jnp.float32),
        scratch_types=[
            pltpu.VMEM((per_w,), jnp.int32),
            pltpu.VMEM((ch, 8, 128), jnp.float32),
            pltpu.VMEM((ch, 8, 128), jnp.float32),
            pltpu.VMEM((ch, 8, 128), jnp.float32),
            pltpu.SemaphoreType.DMA,
            pltpu.SemaphoreType.DMA,
            pltpu.SemaphoreType.DMA,
            pltpu.SemaphoreType.DMA,
            pltpu.SemaphoreType.DMA,
            pltpu.SemaphoreType.DMA,
        ],
    )
    def gather_kernel(table_hbm, idx_hbm, out_hbm, idx_all, buf0, buf1, buf2,
                      gsem0, gsem1, gsem2, wsem0, wsem1, wsem2):
        wid = lax.axis_index("s") * _NC + lax.axis_index("c")
        base = wid * per_w
        buf_b = [buf0, buf1, buf2]
        gsem_b = [gsem0, gsem1, gsem2]
        wsem_b = [wsem0, wsem1, wsem2]

        pltpu.sync_copy(idx_hbm.at[pl.ds(base, per_w)], idx_all)

        for c in range(min(3, n_ch)):
            pltpu.async_copy(table_hbm.at[idx_all.at[pl.ds(c * ch, ch)]],
                             buf_b[c % 3], gsem_b[c % 3])
        for c in range(n_ch):
            p = c % 3
            pltpu.make_async_copy(
                table_hbm.at[idx_all.at[pl.ds(c * ch, ch)]],
                buf_b[p], gsem_b[p]).wait()
            pltpu.async_copy(buf_b[p], out_hbm.at[pl.ds(base + c * ch, ch)],
                             wsem_b[p])
            if c + 3 < n_ch:
                pltpu.make_async_copy(buf_b[p], out_hbm.at[pl.ds(0, ch)],
                                      wsem_b[p]).wait()
                pltpu.async_copy(
                    table_hbm.at[idx_all.at[pl.ds((c + 3) * ch, ch)]],
                    buf_b[p], gsem_b[p])
        for p in range(min(3, n_ch)):
            pltpu.make_async_copy(buf_b[p], out_hbm.at[pl.ds(0, ch)],
                                  wsem_b[p]).wait()

    return gather_kernel(table, idx)


def _tc_scan(g, s_in, fin, batch, length, s):
    d = s * s
    o = fin.shape[1]

    def scan_kernel(sin_ref, fin_ref, g_ref, r_ref, sfin_ref, s_ref):
        l = pl.program_id(0)

        @pl.when(l == 0)
        def _():
            s_ref[...] = sin_ref[...]

        st = s_ref[...]
        gl = jnp.reshape(g_ref[...], (batch, d))

        row = lax.broadcasted_iota(jnp.int32, (2 * s, d), 0) % s
        colk = lax.broadcasted_iota(jnp.int32, (2 * s, d), 1) // s
        e2 = jnp.where(row == colk, 1.0, 0.0).astype(jnp.float32)
        st_hi = st.astype(jnp.bfloat16).astype(jnp.float32)
        st_lo = st - st_hi
        st2 = jnp.concatenate([st_hi, st_lo], axis=1)
        se = jax.lax.dot_general(
            st2, e2, (((1,), (0,)), ((), ())),
            preferred_element_type=jnp.float32)

        w = gl * se

        w128 = w[:, 0:128]
        for c in range(1, d // 128):
            w128 = w128 + w[:, c * 128:(c + 1) * 128]
        s_new = w128[:, 0:s]
        for q in range(1, 128 // s):
            s_new = s_new + w128[:, q * s:(q + 1) * s]

        s_ref[...] = s_new
        r_ref[0] = jax.lax.dot_general(
            s_new, fin_ref[...], (((1,), (0,)), ((), ())),
            precision=lax.Precision.HIGHEST,
            preferred_element_type=jnp.float32)

        @pl.when(l == length - 1)
        def _():
            sfin_ref[...] = s_new

    return pl.pallas_call(
        scan_kernel,
        grid=(length,),
        in_specs=[
            pl.BlockSpec((batch, s), lambda l: (0, 0)),
            pl.BlockSpec((s, o), lambda l: (0, 0)),
            pl.BlockSpec((batch // 8, 8, 8, 128), lambda l: (l, 0, 0, 0)),
        ],
        out_specs=[
            pl.BlockSpec((1, batch, o), lambda l: (l, 0, 0)),
            pl.BlockSpec((batch, s), lambda l: (0, 0)),
        ],
        out_shape=[
            jax.ShapeDtypeStruct((length, batch, o), jnp.float32),
            jax.ShapeDtypeStruct((batch, s), jnp.float32),
        ],
        scratch_shapes=[pltpu.VMEM((batch, s), jnp.float32)],
        compiler_params=pltpu.CompilerParams(
            dimension_semantics=("arbitrary",)),
    )(s_in, fin, g)


def kernel(action_seq, trans_prob, fin_matrix):
    batch, length = action_seq.shape
    a, s, _ = trans_prob.shape
    d = s * s
    n = length * batch

    table3 = jnp.reshape(trans_prob, (a, 8, d // 8))
    idx = jnp.reshape(jnp.transpose(action_seq, (1, 0)),
                      (n,)).astype(jnp.int32)

    lcs = [10] * 5 if length == 50 else [length]

    starts = [sum(lcs[:c]) for c in range(len(lcs))]
    gs = [_sc_gather(table3,
                     lax.slice(idx, (st0 * batch,), ((st0 + lc) * batch,)),
                     lc * batch)
          for st0, lc in zip(starts, lcs)]

    st = jnp.zeros((batch, s), jnp.float32).at[:, 0].set(1.0)
    parts = []
    for g_c, lc in zip(gs, lcs):
        g4 = jnp.reshape(g_c, (lc * batch // 8, 8, 8, d // 8))
        rewards_c, st = _tc_scan(g4, st, fin_matrix, batch, lc, s)
        parts.append(rewards_c)

    rewards = jnp.transpose(jnp.concatenate(parts, axis=0), (1, 0, 2))
    return rewards, st
